# baseline (speedup 1.0000x reference)
import jax
import jax.numpy as jnp
from jax import lax
from jax.experimental import pallas as pl
from jax.experimental.pallas import tpu as pltpu
from jax.experimental.pallas import tpu_sc as plsc

NUM_VOCAB = 49408
NUM_EMBED = 768
NUM_TOKENS = 77
BATCH = 1024

_NW = 32
_ITEMS_PER_W = BATCH // _NW
_PAIRS = _ITEMS_PER_W // 2
_TPAD = 80
_H1 = 40
_H2 = 32
_MAIN = _H1 + _H2
_TAIL = NUM_TOKENS - _MAIN
_LANES = 16
_CPL = NUM_EMBED // _LANES


def _add_rows(buf, pos_v, lo, hi):
  def row_body(r, carry):
    for c in range(_CPL):
      sl = pl.ds(c * _LANES, _LANES)
      plsc.addupdate(buf.at[r, sl], pos_v[r, sl])
    return carry

  lax.fori_loop(lo, hi, row_body, 0, unroll=False)


def _embed_body(table_hbm, idx_hbm, pos_hbm, out_hbm,
                idx0, idx1, pos_v, buf, tail_v,
                g1, g2, ts, ss, i0s, i1s):
  idxs = (idx0, idx1)
  isems = (i0s, i1s)

  wid = lax.axis_index("s") * 2 + lax.axis_index("c")
  item_base = wid * _ITEMS_PER_W

  pltpu.sync_copy(pos_hbm, pos_v)
  pltpu.sync_copy(idx_hbm.at[pl.ds(pl.multiple_of(item_base * _TPAD, 8),
                                   _TPAD)], idx0)

  def process(i, s):
    idx_v = idxs[s]

    @pl.when(i > 0)
    def _():
      pltpu.make_async_copy(buf, out_hbm.at[0], ss).wait()

    h1 = pltpu.async_copy(table_hbm.at[idx_v.at[pl.ds(0, _H1)]],
                          buf.at[pl.ds(0, _H1)], g1)
    h2 = pltpu.async_copy(table_hbm.at[idx_v.at[pl.ds(_H1, _H2)]],
                          buf.at[pl.ds(_H1, _H2)], g2)
    ht = pltpu.async_copy(table_hbm.at[idx_v.at[pl.ds(_MAIN, 8)]],
                          tail_v, ts)

    nxt = i + 1

    @pl.when(nxt < _ITEMS_PER_W)
    def _():
      pltpu.async_copy(
          idx_hbm.at[pl.ds(pl.multiple_of((item_base + nxt) * _TPAD, 8),
                           _TPAD)], idxs[1 - s], isems[1 - s])

    h1.wait()
    _add_rows(buf, pos_v, 0, _H1)
    ht.wait()

    def tail_body(r, carry):
      for c in range(_CPL):
        sl = pl.ds(c * _LANES, _LANES)
        buf[_MAIN + r, sl] = tail_v[r, sl]
      return carry

    lax.fori_loop(0, _TAIL, tail_body, 0, unroll=False)
    h2.wait()
    _add_rows(buf, pos_v, _H1, NUM_TOKENS)

    pltpu.async_copy(buf, out_hbm.at[item_base + i], ss)

    @pl.when(nxt < _ITEMS_PER_W)
    def _():
      pltpu.make_async_copy(
          idx_hbm.at[pl.ds(0, _TPAD)], idxs[1 - s], isems[1 - s]).wait()

  def pair_body(p, carry):
    process(2 * p, 0)
    process(2 * p + 1, 1)
    return carry

  lax.fori_loop(0, _PAIRS, pair_body, 0, unroll=False)
  pltpu.make_async_copy(buf, out_hbm.at[0], ss).wait()


@jax.jit
def _embed(table, idx, pos):
  mesh = plsc.VectorSubcoreMesh(core_axis_name="c", subcore_axis_name="s",
                                num_cores=2, num_subcores=16)
  return pl.kernel(
      _embed_body,
      out_type=jax.ShapeDtypeStruct((BATCH, NUM_TOKENS, NUM_EMBED),
                                    jnp.float32),
      mesh=mesh,
      scratch_types=[
          pltpu.VMEM((_TPAD,), jnp.int32),
          pltpu.VMEM((_TPAD,), jnp.int32),
          pltpu.VMEM((NUM_TOKENS, NUM_EMBED), jnp.float32),
          pltpu.VMEM((NUM_TOKENS, NUM_EMBED), jnp.float32),
          pltpu.VMEM((8, NUM_EMBED), jnp.float32),
      ] + [pltpu.SemaphoreType.DMA] * 6,
  )(table, idx, pos)


def kernel(tokens, token_embedding, positional_embedding):
  idx = jnp.pad(tokens.astype(jnp.int32),
                ((0, 0), (0, _TPAD - NUM_TOKENS))).reshape(-1)
  return _embed(token_embedding, idx, positional_embedding)


# speedup vs baseline: 1.0312x; 1.0312x over previous
import jax
import jax.numpy as jnp
from jax import lax
from jax.experimental import pallas as pl
from jax.experimental.pallas import tpu as pltpu
from jax.experimental.pallas import tpu_sc as plsc

NUM_VOCAB = 49408
NUM_EMBED = 768
NUM_TOKENS = 77
BATCH = 1024

_NW = 32
_ITEMS_PER_W = BATCH // _NW
_PAIRS = _ITEMS_PER_W // 2
_TPAD = 80
_H1 = 40
_H2 = 32
_MAIN = _H1 + _H2
_TAIL = NUM_TOKENS - _MAIN
_LANES = 16
_CPL = NUM_EMBED // _LANES


def _add_rows(buf, pos_v, lo, hi):
  def row_body(r, carry):
    for c in range(_CPL):
      sl = pl.ds(c * _LANES, _LANES)
      plsc.addupdate(buf.at[r, sl], pos_v[r, sl])
    return carry

  lax.fori_loop(lo, hi, row_body, 0, unroll=2)


def _embed_body(table_hbm, idx_hbm, pos_hbm, out_hbm,
                idx0, idx1, pos_v, buf, tail_v,
                g1, g2, ts, ss, i0s, i1s):
  idxs = (idx0, idx1)
  isems = (i0s, i1s)

  wid = lax.axis_index("s") * 2 + lax.axis_index("c")
  item_base = wid * _ITEMS_PER_W

  pltpu.sync_copy(pos_hbm, pos_v)
  pltpu.sync_copy(idx_hbm.at[pl.ds(pl.multiple_of(item_base * _TPAD, 8),
                                   _TPAD)], idx0)

  def process(i, s):
    idx_v = idxs[s]

    @pl.when(i > 0)
    def _():
      pltpu.make_async_copy(buf, out_hbm.at[0], ss).wait()

    h1 = pltpu.async_copy(table_hbm.at[idx_v.at[pl.ds(0, _H1)]],
                          buf.at[pl.ds(0, _H1)], g1)
    h2 = pltpu.async_copy(table_hbm.at[idx_v.at[pl.ds(_H1, _H2)]],
                          buf.at[pl.ds(_H1, _H2)], g2)
    ht = pltpu.async_copy(table_hbm.at[idx_v.at[pl.ds(_MAIN, 8)]],
                          tail_v, ts)

    nxt = i + 1

    @pl.when(nxt < _ITEMS_PER_W)
    def _():
      pltpu.async_copy(
          idx_hbm.at[pl.ds(pl.multiple_of((item_base + nxt) * _TPAD, 8),
                           _TPAD)], idxs[1 - s], isems[1 - s])

    h1.wait()
    _add_rows(buf, pos_v, 0, _H1)
    ht.wait()

    def tail_body(r, carry):
      for c in range(_CPL):
        sl = pl.ds(c * _LANES, _LANES)
        buf[_MAIN + r, sl] = tail_v[r, sl]
      return carry

    lax.fori_loop(0, _TAIL, tail_body, 0, unroll=False)
    h2.wait()
    _add_rows(buf, pos_v, _H1, NUM_TOKENS)

    pltpu.async_copy(buf, out_hbm.at[item_base + i], ss)

    @pl.when(nxt < _ITEMS_PER_W)
    def _():
      pltpu.make_async_copy(
          idx_hbm.at[pl.ds(0, _TPAD)], idxs[1 - s], isems[1 - s]).wait()

  def pair_body(p, carry):
    process(2 * p, 0)
    process(2 * p + 1, 1)
    return carry

  lax.fori_loop(0, _PAIRS, pair_body, 0, unroll=False)
  pltpu.make_async_copy(buf, out_hbm.at[0], ss).wait()


@jax.jit
def _embed(table, idx, pos):
  mesh = plsc.VectorSubcoreMesh(core_axis_name="c", subcore_axis_name="s",
                                num_cores=2, num_subcores=16)
  return pl.kernel(
      _embed_body,
      out_type=jax.ShapeDtypeStruct((BATCH, NUM_TOKENS, NUM_EMBED),
                                    jnp.float32),
      mesh=mesh,
      scratch_types=[
          pltpu.VMEM((_TPAD,), jnp.int32),
          pltpu.VMEM((_TPAD,), jnp.int32),
          pltpu.VMEM((NUM_TOKENS, NUM_EMBED), jnp.float32),
          pltpu.VMEM((NUM_TOKENS, NUM_EMBED), jnp.float32),
          pltpu.VMEM((8, NUM_EMBED), jnp.float32),
      ] + [pltpu.SemaphoreType.DMA] * 6,
  )(table, idx, pos)


def kernel(tokens, token_embedding, positional_embedding):
  idx = jnp.pad(tokens.astype(jnp.int32),
                ((0, 0), (0, _TPAD - NUM_TOKENS))).reshape(-1)
  return _embed(token_embedding, idx, positional_embedding)


# speedup vs baseline: 1.0318x; 1.0005x over previous
import jax
import jax.numpy as jnp
from jax import lax
from jax.experimental import pallas as pl
from jax.experimental.pallas import tpu as pltpu
from jax.experimental.pallas import tpu_sc as plsc

NUM_VOCAB = 49408
NUM_EMBED = 768
NUM_TOKENS = 77
BATCH = 1024

_NW = 32
_ITEMS_PER_W = BATCH // _NW
_PAIRS = _ITEMS_PER_W // 2
_TPAD = 80
_H1 = 40
_H2 = 32
_MAIN = _H1 + _H2
_TAIL = NUM_TOKENS - _MAIN
_LANES = 16
_CPL = NUM_EMBED // _LANES


def _add_rows(buf, pos_v, lo, hi):
  def row_body(r, carry):
    for c in range(_CPL):
      sl = pl.ds(c * _LANES, _LANES)
      plsc.addupdate(buf.at[r, sl], pos_v[r, sl])
    return carry

  lax.fori_loop(lo, hi, row_body, 0, unroll=4)


def _embed_body(table_hbm, idx_hbm, pos_hbm, out_hbm,
                idx0, idx1, pos_v, buf, tail_v,
                g1, g2, ts, ss, i0s, i1s):
  idxs = (idx0, idx1)
  isems = (i0s, i1s)

  wid = lax.axis_index("s") * 2 + lax.axis_index("c")
  item_base = wid * _ITEMS_PER_W

  pltpu.sync_copy(pos_hbm, pos_v)
  pltpu.sync_copy(idx_hbm.at[pl.ds(pl.multiple_of(item_base * _TPAD, 8),
                                   _TPAD)], idx0)

  def process(i, s):
    idx_v = idxs[s]

    @pl.when(i > 0)
    def _():
      pltpu.make_async_copy(buf, out_hbm.at[0], ss).wait()

    h1 = pltpu.async_copy(table_hbm.at[idx_v.at[pl.ds(0, _H1)]],
                          buf.at[pl.ds(0, _H1)], g1)
    h2 = pltpu.async_copy(table_hbm.at[idx_v.at[pl.ds(_H1, _H2)]],
                          buf.at[pl.ds(_H1, _H2)], g2)
    ht = pltpu.async_copy(table_hbm.at[idx_v.at[pl.ds(_MAIN, 8)]],
                          tail_v, ts)

    nxt = i + 1

    @pl.when(nxt < _ITEMS_PER_W)
    def _():
      pltpu.async_copy(
          idx_hbm.at[pl.ds(pl.multiple_of((item_base + nxt) * _TPAD, 8),
                           _TPAD)], idxs[1 - s], isems[1 - s])

    h1.wait()
    _add_rows(buf, pos_v, 0, _H1)
    ht.wait()

    def tail_body(r, carry):
      for c in range(_CPL):
        sl = pl.ds(c * _LANES, _LANES)
        buf[_MAIN + r, sl] = tail_v[r, sl]
      return carry

    lax.fori_loop(0, _TAIL, tail_body, 0, unroll=False)
    h2.wait()
    _add_rows(buf, pos_v, _H1, NUM_TOKENS)

    pltpu.async_copy(buf, out_hbm.at[item_base + i], ss)

    @pl.when(nxt < _ITEMS_PER_W)
    def _():
      pltpu.make_async_copy(
          idx_hbm.at[pl.ds(0, _TPAD)], idxs[1 - s], isems[1 - s]).wait()

  def pair_body(p, carry):
    process(2 * p, 0)
    process(2 * p + 1, 1)
    return carry

  lax.fori_loop(0, _PAIRS, pair_body, 0, unroll=False)
  pltpu.make_async_copy(buf, out_hbm.at[0], ss).wait()


@jax.jit
def _embed(table, idx, pos):
  mesh = plsc.VectorSubcoreMesh(core_axis_name="c", subcore_axis_name="s",
                                num_cores=2, num_subcores=16)
  return pl.kernel(
      _embed_body,
      out_type=jax.ShapeDtypeStruct((BATCH, NUM_TOKENS, NUM_EMBED),
                                    jnp.float32),
      mesh=mesh,
      scratch_types=[
          pltpu.VMEM((_TPAD,), jnp.int32),
          pltpu.VMEM((_TPAD,), jnp.int32),
          pltpu.VMEM((NUM_TOKENS, NUM_EMBED), jnp.float32),
          pltpu.VMEM((NUM_TOKENS, NUM_EMBED), jnp.float32),
          pltpu.VMEM((8, NUM_EMBED), jnp.float32),
      ] + [pltpu.SemaphoreType.DMA] * 6,
  )(table, idx, pos)


def kernel(tokens, token_embedding, positional_embedding):
  idx = jnp.pad(tokens.astype(jnp.int32),
                ((0, 0), (0, _TPAD - NUM_TOKENS))).reshape(-1)
  return _embed(token_embedding, idx, positional_embedding)


# speedup vs baseline: 1.0327x; 1.0009x over previous
import jax
import jax.numpy as jnp
from jax import lax
from jax.experimental import pallas as pl
from jax.experimental.pallas import tpu as pltpu
from jax.experimental.pallas import tpu_sc as plsc

NUM_VOCAB = 49408
NUM_EMBED = 768
NUM_TOKENS = 77
BATCH = 1024

_NW = 32
_ITEMS_PER_W = BATCH // _NW
_PAIRS = _ITEMS_PER_W // 2
_TPAD = 80
_H1 = 40
_H2 = 24
_H3 = 8
_MAIN = _H1 + _H2 + _H3
_TAIL = NUM_TOKENS - _MAIN
_LANES = 16
_CPL = NUM_EMBED // _LANES


def _add_rows(buf, pos_v, lo, hi):
  def row_body(r, carry):
    for c in range(_CPL):
      sl = pl.ds(c * _LANES, _LANES)
      plsc.addupdate(buf.at[r, sl], pos_v[r, sl])
    return carry

  lax.fori_loop(lo, hi, row_body, 0, unroll=2)


def _embed_body(table_hbm, idx_hbm, pos_hbm, out_hbm,
                idx0, idx1, pos_v, buf, tail_v,
                g1, g2, g3, ts, ss, i0s, i1s):
  idxs = (idx0, idx1)
  isems = (i0s, i1s)

  wid = lax.axis_index("s") * 2 + lax.axis_index("c")
  item_base = wid * _ITEMS_PER_W

  pltpu.sync_copy(pos_hbm, pos_v)
  pltpu.sync_copy(idx_hbm.at[pl.ds(pl.multiple_of(item_base * _TPAD, 8),
                                   _TPAD)], idx0)

  def process(i, s):
    idx_v = idxs[s]

    @pl.when(i > 0)
    def _():
      pltpu.make_async_copy(buf, out_hbm.at[0], ss).wait()

    h1 = pltpu.async_copy(table_hbm.at[idx_v.at[pl.ds(0, _H1)]],
                          buf.at[pl.ds(0, _H1)], g1)
    h2 = pltpu.async_copy(table_hbm.at[idx_v.at[pl.ds(_H1, _H2)]],
                          buf.at[pl.ds(_H1, _H2)], g2)
    h3 = pltpu.async_copy(table_hbm.at[idx_v.at[pl.ds(_H1 + _H2, _H3)]],
                          buf.at[pl.ds(_H1 + _H2, _H3)], g3)
    ht = pltpu.async_copy(table_hbm.at[idx_v.at[pl.ds(_MAIN, 8)]],
                          tail_v, ts)

    nxt = i + 1

    @pl.when(nxt < _ITEMS_PER_W)
    def _():
      pltpu.async_copy(
          idx_hbm.at[pl.ds(pl.multiple_of((item_base + nxt) * _TPAD, 8),
                           _TPAD)], idxs[1 - s], isems[1 - s])

    h1.wait()
    _add_rows(buf, pos_v, 0, _H1)
    h2.wait()
    _add_rows(buf, pos_v, _H1, _H1 + _H2)
    ht.wait()

    def tail_body(r, carry):
      for c in range(_CPL):
        sl = pl.ds(c * _LANES, _LANES)
        buf[_MAIN + r, sl] = tail_v[r, sl]
      return carry

    lax.fori_loop(0, _TAIL, tail_body, 0, unroll=True)
    h3.wait()
    _add_rows(buf, pos_v, _H1 + _H2, NUM_TOKENS)

    pltpu.async_copy(buf, out_hbm.at[item_base + i], ss)

    @pl.when(nxt < _ITEMS_PER_W)
    def _():
      pltpu.make_async_copy(
          idx_hbm.at[pl.ds(0, _TPAD)], idxs[1 - s], isems[1 - s]).wait()

  def pair_body(p, carry):
    process(2 * p, 0)
    process(2 * p + 1, 1)
    return carry

  lax.fori_loop(0, _PAIRS, pair_body, 0, unroll=False)
  pltpu.make_async_copy(buf, out_hbm.at[0], ss).wait()


@jax.jit
def _embed(table, idx, pos):
  mesh = plsc.VectorSubcoreMesh(core_axis_name="c", subcore_axis_name="s",
                                num_cores=2, num_subcores=16)
  return pl.kernel(
      _embed_body,
      out_type=jax.ShapeDtypeStruct((BATCH, NUM_TOKENS, NUM_EMBED),
                                    jnp.float32),
      mesh=mesh,
      scratch_types=[
          pltpu.VMEM((_TPAD,), jnp.int32),
          pltpu.VMEM((_TPAD,), jnp.int32),
          pltpu.VMEM((NUM_TOKENS, NUM_EMBED), jnp.float32),
          pltpu.VMEM((NUM_TOKENS, NUM_EMBED), jnp.float32),
          pltpu.VMEM((8, NUM_EMBED), jnp.float32),
      ] + [pltpu.SemaphoreType.DMA] * 7,
  )(table, idx, pos)


def kernel(tokens, token_embedding, positional_embedding):
  idx = jnp.pad(tokens.astype(jnp.int32),
                ((0, 0), (0, _TPAD - NUM_TOKENS))).reshape(-1)
  return _embed(token_embedding, idx, positional_embedding)
